# Initial kernel scaffold; baseline (speedup 1.0000x reference)
import functools
import jax
import jax.numpy as jnp
from jax import lax
from jax.experimental import pallas as pl
from jax.experimental.pallas import tpu as pltpu
from jax.experimental.pallas import tpu_sc as plsc

N = 10000
E = 320000
IN = 128
H = 4
C = 32
HID = H * C
NEG = 0.2

NPAD = 10240
NC = 2
NS = 16
NW = NC * NS
EPW = E // NW
B = 80
NCHUNK = EPW // B
RPT = NPAD // NS
DL = 16
BLK = 1024
NBLK = NPAD // BLK


def _proj_body(x_ref, w_ref, aw_ref, proj_ref, a_ref):
    p = jnp.dot(x_ref[...], w_ref[...], preferred_element_type=jnp.float32)
    proj_ref[...] = p
    a_ref[...] = jnp.dot(p, aw_ref[...], preferred_element_type=jnp.float32)


def _final_body(acc_ref, den_ref, proj_ref, a_ref, er4_ref, er16_ref, b_ref,
                o_ref):
    a = a_ref[...]
    l = a[:, :H] + a[:, H:]
    l = jnp.where(l >= 0.0, l, l * NEG)
    ews = jnp.exp(l)
    ews_e = jnp.dot(ews, er4_ref[...], preferred_element_type=jnp.float32)
    den = den_ref[0] + den_ref[1]
    den_e = jnp.dot(den, er16_ref[...], preferred_element_type=jnp.float32)
    den_e = den_e + ews_e
    acc = acc_ref[0] + acc_ref[1] + ews_e * proj_ref[...]
    o_ref[...] = acc / den_e + b_ref[...]


_sc_mesh = plsc.VectorSubcoreMesh(core_axis_name="c", subcore_axis_name="s")


@functools.partial(
    pl.kernel,
    out_type=(
        jax.ShapeDtypeStruct((NC, NPAD, HID), jnp.float32),
        jax.ShapeDtypeStruct((NC, NPAD, DL), jnp.float32),
    ),
    mesh=_sc_mesh,
    scratch_types=[
        pltpu.VMEM((2 * H, NPAD), jnp.float32),
        pltpu.VMEM((B, HID), jnp.float32),
        pltpu.VMEM((B,), jnp.int32),
        pltpu.VMEM((B,), jnp.int32),
        pltpu.VMEM((B, DL), jnp.float32),
        pltpu.VMEM_SHARED((NPAD, HID), jnp.float32),
        pltpu.VMEM_SHARED((NPAD, DL), jnp.float32),
        pltpu.SemaphoreType.DMA,
    ],
)
def _sc_gat(proj_hbm, at_hbm, row_hbm, col_hbm, acc_hbm, den_hbm,
            atab, rows, ridx, cidx, ew, acc_sh, den_sh, sem):
    c = lax.axis_index("c")
    s = lax.axis_index("s")
    wid = c * NS + s
    ebase = wid * EPW

    pltpu.sync_copy(at_hbm, atab)

    zf = jnp.zeros((16,), jnp.float32)

    def _zero(i, carry):
        for v in range(HID // 16):
            rows[i, pl.ds(v * 16, 16)] = zf
        ew[i, pl.ds(0, 16)] = zf
        return carry

    lax.fori_loop(0, B, _zero, 0)
    rbase = s * RPT
    for k in range(RPT // B):
        pltpu.sync_copy(rows, acc_sh.at[pl.ds(rbase + k * B, B)])
        pltpu.sync_copy(ew, den_sh.at[pl.ds(rbase + k * B, B)])
    plsc.subcore_barrier()

    iota16 = lax.iota(jnp.int32, 16)

    def _chunk(t, carry):
        base = ebase + t * B
        pltpu.sync_copy(row_hbm.at[pl.ds(base, B)], ridx)
        pltpu.sync_copy(col_hbm.at[pl.ds(base, B)], cidx)
        gat = pltpu.async_copy(proj_hbm.at[ridx], rows, sem)

        def _grp(g, carry2):
            rv = ridx[pl.ds(g * 16, 16)]
            cv = cidx[pl.ds(g * 16, 16)]
            erow = iota16 + g * 16
            valid = rv != cv
            for h in range(H):
                hsrc = jnp.full((16,), h, jnp.int32)
                hdst = jnp.full((16,), h + H, jnp.int32)
                asrc = plsc.load_gather(atab, [hsrc, rv])
                adst = plsc.load_gather(atab, [hdst, cv])
                l = asrc + adst
                l = jnp.where(l >= 0.0, l, l * NEG)
                w = jnp.exp(l)
                w = jnp.where(valid, w, 0.0)
                plsc.store_scatter(ew, [erow, hsrc], w)
            return carry2

        lax.fori_loop(0, B // 16, _grp, 0)
        gat.wait()

        def _scale(e, carry2):
            for h in range(H):
                wsc = ew[e, h]
                for v in range(2 * h, 2 * h + 2):
                    rows[e, pl.ds(v * 16, 16)] = rows[e, pl.ds(v * 16, 16)] * wsc
            return carry2

        lax.fori_loop(0, B, _scale, 0)

        pltpu.sync_copy(rows, acc_sh.at[cidx], add=True)
        pltpu.sync_copy(ew, den_sh.at[cidx], add=True)
        return carry

    lax.fori_loop(0, NCHUNK, _chunk, 0)

    plsc.subcore_barrier()
    pltpu.sync_copy(acc_sh.at[pl.ds(rbase, RPT)], acc_hbm.at[c, pl.ds(rbase, RPT)])
    pltpu.sync_copy(den_sh.at[pl.ds(rbase, RPT)], den_hbm.at[c, pl.ds(rbase, RPT)])


@jax.jit
def _run(x, edge_index, W, att_src, att_dst, b):
    xp = jnp.zeros((NPAD, IN), jnp.float32).at[:N].set(x)
    eye4 = jnp.eye(H, dtype=jnp.float32)
    aw_src = (att_src[:, :, None] * eye4[:, None, :]).reshape(HID, H)
    aw_dst = (att_dst[:, :, None] * eye4[:, None, :]).reshape(HID, H)
    attW = jnp.concatenate([aw_src, aw_dst], axis=1)

    proj, a_all = pl.pallas_call(
        _proj_body,
        grid=(NBLK,),
        in_specs=[
            pl.BlockSpec((BLK, IN), lambda i: (i, 0)),
            pl.BlockSpec((IN, HID), lambda i: (0, 0)),
            pl.BlockSpec((HID, 2 * H), lambda i: (0, 0)),
        ],
        out_specs=[
            pl.BlockSpec((BLK, HID), lambda i: (i, 0)),
            pl.BlockSpec((BLK, 2 * H), lambda i: (i, 0)),
        ],
        out_shape=[
            jax.ShapeDtypeStruct((NPAD, HID), jnp.float32),
            jax.ShapeDtypeStruct((NPAD, 2 * H), jnp.float32),
        ],
    )(xp, W, attW)

    a_t = a_all.T
    row = edge_index[0]
    col = edge_index[1]
    acc, den = _sc_gat(proj, a_t, row, col)

    er4 = jnp.repeat(jnp.eye(H, dtype=jnp.float32), C, axis=1)
    er16 = jnp.zeros((DL, HID), jnp.float32).at[:H].set(er4)
    b2 = b.reshape(1, HID)

    out = pl.pallas_call(
        _final_body,
        grid=(NBLK,),
        in_specs=[
            pl.BlockSpec((NC, BLK, HID), lambda i: (0, i, 0)),
            pl.BlockSpec((NC, BLK, DL), lambda i: (0, i, 0)),
            pl.BlockSpec((BLK, HID), lambda i: (i, 0)),
            pl.BlockSpec((BLK, 2 * H), lambda i: (i, 0)),
            pl.BlockSpec((H, HID), lambda i: (0, 0)),
            pl.BlockSpec((DL, HID), lambda i: (0, 0)),
            pl.BlockSpec((1, HID), lambda i: (0, 0)),
        ],
        out_specs=pl.BlockSpec((BLK, HID), lambda i: (i, 0)),
        out_shape=jax.ShapeDtypeStruct((NPAD, HID), jnp.float32),
    )(acc, den, proj, a_all, er4, er16, b2)

    return out[:N]


def kernel(x, edge_index, W, att_src, att_dst, b):
    return _run(x, edge_index, W, att_src, att_dst, b)


# speedup vs baseline: 83.7335x; 83.7335x over previous
import functools
import jax
import jax.numpy as jnp
from jax import lax
from jax.experimental import pallas as pl
from jax.experimental.pallas import tpu as pltpu
from jax.experimental.pallas import tpu_sc as plsc

N = 10000
E = 320000
IN = 128
H = 4
C = 32
HID = H * C
NEG = 0.2

NPAD = 10240
NC = 2
NS = 16
NW = NC * NS
EPW = E // NW
B = 80
NCHUNK = EPW // B
RPT = NPAD // NS
DL = 16
BLK = 1024
NBLK = NPAD // BLK


def _proj_body(x_ref, w_ref, aw1_ref, aw2_ref, proj_ref, at1_ref, at2_ref):
    p = jnp.dot(x_ref[...], w_ref[...], preferred_element_type=jnp.float32)
    proj_ref[...] = p
    at1_ref[...] = jnp.dot(p, aw1_ref[...], preferred_element_type=jnp.float32)
    at2_ref[...] = jnp.dot(p, aw2_ref[...], preferred_element_type=jnp.float32)


def _final_body(acc_ref, den_ref, proj_ref, at1_ref, at2_ref, er4_ref,
                er16_ref, b_ref, o_ref):
    l = at1_ref[:, :H] + at2_ref[:, :H]
    l = jnp.where(l >= 0.0, l, l * NEG)
    ews = jnp.exp(l)
    ews_e = jnp.dot(ews, er4_ref[...], preferred_element_type=jnp.float32)
    den = den_ref[0] + den_ref[1]
    den_e = jnp.dot(den, er16_ref[...], preferred_element_type=jnp.float32)
    den_e = den_e + ews_e
    acc = acc_ref[0] + acc_ref[1] + ews_e * proj_ref[...]
    o_ref[...] = acc / den_e + b_ref[...]


_sc_mesh = plsc.VectorSubcoreMesh(core_axis_name="c", subcore_axis_name="s")


@functools.partial(
    pl.kernel,
    out_type=(
        jax.ShapeDtypeStruct((NC, NPAD, HID), jnp.float32),
        jax.ShapeDtypeStruct((NC, NPAD, DL), jnp.float32),
    ),
    mesh=_sc_mesh,
    compiler_params=pltpu.CompilerParams(use_tc_tiling_on_sc=False,
                                         needs_layout_passes=False),
    scratch_types=[
        pltpu.VMEM((B, HID), jnp.float32),
        pltpu.VMEM((B, DL), jnp.float32),
        pltpu.VMEM((B, DL), jnp.float32),
        pltpu.VMEM((B,), jnp.int32),
        pltpu.VMEM((B,), jnp.int32),
        pltpu.VMEM((B, DL), jnp.float32),
        pltpu.VMEM_SHARED((NPAD, HID), jnp.float32),
        pltpu.VMEM_SHARED((NPAD, DL), jnp.float32),
        pltpu.SemaphoreType.DMA,
        pltpu.SemaphoreType.DMA,
    ],
)
def _sc_gat(proj_hbm, at1_hbm, at2_hbm, row_hbm, col_hbm, acc_hbm, den_hbm,
            rows, a1, a2, ridx, cidx, ew, acc_sh, den_sh, sem, sem2):
    c = lax.axis_index("c")
    s = lax.axis_index("s")
    wid = c * NS + s
    ebase = wid * EPW

    zf = jnp.zeros((16,), jnp.float32)

    def _zrow(i, carry):
        for v in range(HID // 16):
            rows[i, pl.ds(v * 16, 16)] = zf
        ew[i, pl.ds(0, 16)] = zf
        return carry

    lax.fori_loop(0, B, _zrow, 0)

    rbase = s * RPT
    for k in range(RPT // B):
        pltpu.sync_copy(rows, acc_sh.at[pl.ds(rbase + k * B, B)])
        pltpu.sync_copy(ew, den_sh.at[pl.ds(rbase + k * B, B)])
    plsc.subcore_barrier()

    lanemask = (lax.iota(jnp.int32, 16) < H).astype(jnp.float32)

    def _chunk(t, carry):
        base = ebase + t * B
        pltpu.sync_copy(row_hbm.at[pl.ds(base, B)], ridx)
        pltpu.sync_copy(col_hbm.at[pl.ds(base, B)], cidx)
        gat = pltpu.async_copy(proj_hbm.at[ridx], rows, sem)
        ga1 = pltpu.async_copy(at1_hbm.at[ridx], a1, sem2)
        ga2 = pltpu.async_copy(at2_hbm.at[cidx], a2, sem2)

        ga1.wait()
        ga2.wait()

        def _grp(g, carry2):
            rv = ridx[pl.ds(g * 16, 16)]
            cv = cidx[pl.ds(g * 16, 16)]
            validf = jnp.where(rv != cv, 1.0, 0.0)

            for j in range(16):
                e = g * 16 + j
                l = a1[e, pl.ds(0, 16)] + a2[e, pl.ds(0, 16)]
                l = jnp.where(l >= 0.0, l, l * NEG)
                w = jnp.exp(l) * validf[j] * lanemask
                ew[e, pl.ds(0, 16)] = w
            return carry2

        lax.fori_loop(0, B // 16, _grp, 0)
        gat.wait()

        def _scale(e, carry2):
            wv = ew[e, pl.ds(0, 16)]
            for h in range(H):
                wsc = wv[h]
                for v in range(2 * h, 2 * h + 2):
                    rows[e, pl.ds(v * 16, 16)] = (
                        rows[e, pl.ds(v * 16, 16)] * wsc)
            return carry2

        lax.fori_loop(0, B, _scale, 0)

        pltpu.sync_copy(rows, acc_sh.at[cidx], add=True)
        pltpu.sync_copy(ew, den_sh.at[cidx], add=True)
        return carry

    lax.fori_loop(0, NCHUNK, _chunk, 0)

    plsc.subcore_barrier()
    pltpu.sync_copy(acc_sh.at[pl.ds(rbase, RPT)], acc_hbm.at[c, pl.ds(rbase, RPT)])
    pltpu.sync_copy(den_sh.at[pl.ds(rbase, RPT)], den_hbm.at[c, pl.ds(rbase, RPT)])


@jax.jit
def _run(x, edge_index, W, att_src, att_dst, b):
    xp = jnp.zeros((NPAD, IN), jnp.float32).at[:N].set(x)
    eye4 = jnp.eye(H, dtype=jnp.float32)
    aw_src = (att_src[:, :, None] * eye4[:, None, :]).reshape(HID, H)
    aw_dst = (att_dst[:, :, None] * eye4[:, None, :]).reshape(HID, H)
    zpad = jnp.zeros((HID, DL - H), jnp.float32)
    aw1 = jnp.concatenate([aw_src, zpad], axis=1)
    aw2 = jnp.concatenate([aw_dst, zpad], axis=1)

    proj, at1, at2 = pl.pallas_call(
        _proj_body,
        grid=(NBLK,),
        in_specs=[
            pl.BlockSpec((BLK, IN), lambda i: (i, 0)),
            pl.BlockSpec((IN, HID), lambda i: (0, 0)),
            pl.BlockSpec((HID, DL), lambda i: (0, 0)),
            pl.BlockSpec((HID, DL), lambda i: (0, 0)),
        ],
        out_specs=[
            pl.BlockSpec((BLK, HID), lambda i: (i, 0)),
            pl.BlockSpec((BLK, DL), lambda i: (i, 0)),
            pl.BlockSpec((BLK, DL), lambda i: (i, 0)),
        ],
        out_shape=[
            jax.ShapeDtypeStruct((NPAD, HID), jnp.float32),
            jax.ShapeDtypeStruct((NPAD, DL), jnp.float32),
            jax.ShapeDtypeStruct((NPAD, DL), jnp.float32),
        ],
    )(xp, W, aw1, aw2)

    row = edge_index[0]
    col = edge_index[1]
    acc, den = _sc_gat(proj, at1, at2, row, col)

    er4 = jnp.repeat(jnp.eye(H, dtype=jnp.float32), C, axis=1)
    er16 = jnp.zeros((DL, HID), jnp.float32).at[:H].set(er4)
    b2 = b.reshape(1, HID)

    out = pl.pallas_call(
        _final_body,
        grid=(NBLK,),
        in_specs=[
            pl.BlockSpec((NC, BLK, HID), lambda i: (0, i, 0)),
            pl.BlockSpec((NC, BLK, DL), lambda i: (0, i, 0)),
            pl.BlockSpec((BLK, HID), lambda i: (i, 0)),
            pl.BlockSpec((BLK, DL), lambda i: (i, 0)),
            pl.BlockSpec((BLK, DL), lambda i: (i, 0)),
            pl.BlockSpec((H, HID), lambda i: (0, 0)),
            pl.BlockSpec((DL, HID), lambda i: (0, 0)),
            pl.BlockSpec((1, HID), lambda i: (0, 0)),
        ],
        out_specs=pl.BlockSpec((BLK, HID), lambda i: (i, 0)),
        out_shape=jax.ShapeDtypeStruct((NPAD, HID), jnp.float32),
    )(acc, den, proj, at1, at2, er4, er16, b2)

    return out[:N]


def kernel(x, edge_index, W, att_src, att_dst, b):
    return _run(x, edge_index, W, att_src, att_dst, b)
